# baseline (speedup 1.0000x reference)
import jax
import jax.numpy as jnp
from jax import lax
from jax.experimental import pallas as pl
from jax.experimental.pallas import tpu as pltpu
from jax.experimental.pallas import tpu_sc as plsc

N_NODE = 1000000
EMB_DIM = 64
BATCH = 16384
PAIR = 2 * EMB_DIM
NROWS2 = N_NODE // 2

NC = 2
NS = 16
L = 16
NW = NC * NS
B_PER_W = BATCH // NW
CHUNK = 128
N_CHUNKS = B_PER_W // CHUNK
GPC = CHUNK // L


def _sc_body(nid_hbm, nbr_hbm, emb2_hbm, bias_hbm,
             oa_hbm, ob_hbm, op_hbm,
             idx_a, idx_b, half_a, half_b,
             pair_a, pair_b, cols_a, cols_b,
             bias_v, acc_v, prob_v, sem):
    wid = lax.axis_index("s") * NC + lax.axis_index("c")
    base = wid * B_PER_W

    pltpu.sync_copy(nid_hbm.at[pl.ds(base, B_PER_W)], idx_a)
    pltpu.sync_copy(nbr_hbm.at[pl.ds(base, B_PER_W)], idx_b)

    def halve(i, _):
        sl = pl.ds(i * L, L)
        half_a[sl] = lax.shift_right_logical(idx_a[sl], 1)
        half_b[sl] = lax.shift_right_logical(idx_b[sl], 1)
        return 0
    lax.fori_loop(0, B_PER_W // L, halve, 0)

    bias_copies = []
    for j in range(N_CHUNKS):
        sl = pl.ds(j * CHUNK, CHUNK)
        bias_copies.append(
            pltpu.async_copy(bias_hbm.at[idx_b.at[sl]], bias_v.at[sl], sem))

    lane = lax.iota(jnp.int32, L)

    def zero(i, _):
        acc_v[pl.ds(i * L, L)] = jnp.zeros((L,), jnp.float32)
        return 0
    lax.fori_loop(0, B_PER_W // L, zero, 0)

    def chunk_body(j, _):
        sl = pl.ds(j * CHUNK, CHUNK)
        ca = pltpu.async_copy(emb2_hbm.at[half_a.at[sl]], pair_a, sem)
        cb = pltpu.async_copy(emb2_hbm.at[half_b.at[sl]], pair_b, sem)
        ca.wait()
        cb.wait()

        def dim_body(d, _):
            def grp(g, _):
                i0 = j * CHUNK + g * L
                ia = idx_a[pl.ds(i0, L)]
                ib = idx_b[pl.ds(i0, L)]
                rows = g * L + lane
                va = plsc.load_gather(pair_a, [rows, (ia & 1) * EMB_DIM + d])
                vb = plsc.load_gather(pair_b, [rows, (ib & 1) * EMB_DIM + d])
                cols_a[d, pl.ds(i0, L)] = va
                cols_b[d, pl.ds(i0, L)] = vb
                acc_v[pl.ds(i0, L)] = acc_v[pl.ds(i0, L)] + va * vb
                return 0
            lax.fori_loop(0, GPC, grp, 0)
            return 0
        lax.fori_loop(0, EMB_DIM, dim_body, 0)
        return 0

    lax.fori_loop(0, N_CHUNKS, chunk_body, 0)

    for c in bias_copies:
        c.wait()

    def prob_grp(g, _):
        sl = pl.ds(g * L, L)
        score = acc_v[sl] + bias_v[sl]
        p = 1.0 / (1.0 + jnp.exp(-score))
        prob_v[sl] = jnp.minimum(jnp.maximum(p, 1e-5), 1.0)
        return 0
    lax.fori_loop(0, B_PER_W // L, prob_grp, 0)

    dst = pl.ds(base, B_PER_W)
    pltpu.sync_copy(cols_a, oa_hbm.at[:, dst])
    pltpu.sync_copy(cols_b, ob_hbm.at[:, dst])
    pltpu.sync_copy(prob_v, op_hbm.at[dst])


def _build():
    mesh = plsc.VectorSubcoreMesh(core_axis_name="c", subcore_axis_name="s")
    return pl.kernel(
        _sc_body,
        out_type=(
            jax.ShapeDtypeStruct((EMB_DIM, BATCH), jnp.float32),
            jax.ShapeDtypeStruct((EMB_DIM, BATCH), jnp.float32),
            jax.ShapeDtypeStruct((BATCH,), jnp.float32),
        ),
        mesh=mesh,
        scratch_types=[
            pltpu.VMEM((B_PER_W,), jnp.int32),
            pltpu.VMEM((B_PER_W,), jnp.int32),
            pltpu.VMEM((B_PER_W,), jnp.int32),
            pltpu.VMEM((B_PER_W,), jnp.int32),
            pltpu.VMEM((CHUNK, PAIR), jnp.float32),
            pltpu.VMEM((CHUNK, PAIR), jnp.float32),
            pltpu.VMEM((EMB_DIM, B_PER_W), jnp.float32),
            pltpu.VMEM((EMB_DIM, B_PER_W), jnp.float32),
            pltpu.VMEM((B_PER_W,), jnp.float32),
            pltpu.VMEM((B_PER_W,), jnp.float32),
            pltpu.VMEM((B_PER_W,), jnp.float32),
            pltpu.SemaphoreType.DMA,
        ],
        compiler_params=pltpu.CompilerParams(
            needs_layout_passes=False, use_tc_tiling_on_sc=True),
    )


def kernel(node_id, node_neighbor_id, embedding_matrix, bias_vector):
    k = _build()
    emb2 = embedding_matrix.reshape(NROWS2, PAIR)
    oa_t, ob_t, prob = k(node_id, node_neighbor_id, emb2, bias_vector)
    return oa_t.T, ob_t.T, prob


# speedup vs baseline: 1.1384x; 1.1384x over previous
import jax
import jax.numpy as jnp
from jax import lax
from jax.experimental import pallas as pl
from jax.experimental.pallas import tpu as pltpu
from jax.experimental.pallas import tpu_sc as plsc

N_NODE = 1000000
EMB_DIM = 64
BATCH = 16384
PADW = 2 * EMB_DIM

NC = 2
NS = 16
L = 16
NW = NC * NS
B_PER_W = BATCH // NW
CHUNK = 128
N_CHUNKS = B_PER_W // CHUNK
GPC = CHUNK // L


def _sc_body(nid_hbm, nbr_hbm, embp_hbm, bias_hbm,
             oa_hbm, ob_hbm, op_hbm,
             idx_a, idx_b, pair_a, pair_b, cols_a, cols_b,
             bias_v, acc_v, prob_v, sem):
    wid = lax.axis_index("s") * NC + lax.axis_index("c")
    base = wid * B_PER_W

    pltpu.sync_copy(nid_hbm.at[pl.ds(base, B_PER_W)], idx_a)
    pltpu.sync_copy(nbr_hbm.at[pl.ds(base, B_PER_W)], idx_b)

    bias_copies = []
    for j in range(N_CHUNKS):
        sl = pl.ds(j * CHUNK, CHUNK)
        bias_copies.append(
            pltpu.async_copy(bias_hbm.at[idx_b.at[sl]], bias_v.at[sl], sem))

    lane = lax.iota(jnp.int32, L)

    def zero(i, _):
        acc_v[pl.ds(i * L, L)] = jnp.zeros((L,), jnp.float32)
        return 0
    lax.fori_loop(0, B_PER_W // L, zero, 0)

    def chunk_body(j, _):
        sl = pl.ds(j * CHUNK, CHUNK)
        ca = pltpu.async_copy(embp_hbm.at[idx_a.at[sl]], pair_a, sem)
        cb = pltpu.async_copy(embp_hbm.at[idx_b.at[sl]], pair_b, sem)
        ca.wait()
        cb.wait()

        def dim_body(d, _):
            col = jnp.full((L,), 0, jnp.int32) + d

            def grp(g, _):
                i0 = j * CHUNK + g * L
                rows = g * L + lane
                va = plsc.load_gather(pair_a, [rows, col])
                vb = plsc.load_gather(pair_b, [rows, col])
                cols_a[d, pl.ds(i0, L)] = va
                cols_b[d, pl.ds(i0, L)] = vb
                acc_v[pl.ds(i0, L)] = acc_v[pl.ds(i0, L)] + va * vb
                return 0
            lax.fori_loop(0, GPC, grp, 0)
            return 0
        lax.fori_loop(0, EMB_DIM, dim_body, 0)
        return 0

    lax.fori_loop(0, N_CHUNKS, chunk_body, 0)

    for c in bias_copies:
        c.wait()

    def prob_grp(g, _):
        sl = pl.ds(g * L, L)
        score = acc_v[sl] + bias_v[sl]
        p = 1.0 / (1.0 + jnp.exp(-score))
        prob_v[sl] = jnp.minimum(jnp.maximum(p, 1e-5), 1.0)
        return 0
    lax.fori_loop(0, B_PER_W // L, prob_grp, 0)

    dst = pl.ds(base, B_PER_W)
    pltpu.sync_copy(cols_a, oa_hbm.at[:, dst])
    pltpu.sync_copy(cols_b, ob_hbm.at[:, dst])
    pltpu.sync_copy(prob_v, op_hbm.at[dst])


def _build():
    mesh = plsc.VectorSubcoreMesh(core_axis_name="c", subcore_axis_name="s")
    return pl.kernel(
        _sc_body,
        out_type=(
            jax.ShapeDtypeStruct((EMB_DIM, BATCH), jnp.float32),
            jax.ShapeDtypeStruct((EMB_DIM, BATCH), jnp.float32),
            jax.ShapeDtypeStruct((BATCH,), jnp.float32),
        ),
        mesh=mesh,
        scratch_types=[
            pltpu.VMEM((B_PER_W,), jnp.int32),
            pltpu.VMEM((B_PER_W,), jnp.int32),
            pltpu.VMEM((CHUNK, PADW), jnp.float32),
            pltpu.VMEM((CHUNK, PADW), jnp.float32),
            pltpu.VMEM((EMB_DIM, B_PER_W), jnp.float32),
            pltpu.VMEM((EMB_DIM, B_PER_W), jnp.float32),
            pltpu.VMEM((B_PER_W,), jnp.float32),
            pltpu.VMEM((B_PER_W,), jnp.float32),
            pltpu.VMEM((B_PER_W,), jnp.float32),
            pltpu.SemaphoreType.DMA,
        ],
        compiler_params=pltpu.CompilerParams(
            needs_layout_passes=False, use_tc_tiling_on_sc=True),
    )


def kernel(node_id, node_neighbor_id, embedding_matrix, bias_vector):
    k = _build()
    embp = jnp.pad(embedding_matrix, ((0, 0), (0, PADW - EMB_DIM)))
    oa_t, ob_t, prob = k(node_id, node_neighbor_id, embp, bias_vector)
    return oa_t.T, ob_t.T, prob


# speedup vs baseline: 1.1592x; 1.0182x over previous
import jax
import jax.numpy as jnp
from jax import lax
from jax.experimental import pallas as pl
from jax.experimental.pallas import tpu as pltpu
from jax.experimental.pallas import tpu_sc as plsc

N_NODE = 1000000
EMB_DIM = 64
BATCH = 16384
PADW = 2 * EMB_DIM

NC = 2
NS = 16
L = 16
NW = NC * NS
B_PER_W = BATCH // NW
CHUNK = 128
N_CHUNKS = B_PER_W // CHUNK
GPC = CHUNK // L


def _sc_body(nid_hbm, nbr_hbm, embp_hbm, bias_hbm,
             oa_hbm, ob_hbm, op_hbm,
             idx_a, idx_b, pair_a, pair_b, cols_a, cols_b,
             bias_v, acc_v, prob_v, sem):
    wid = lax.axis_index("s") * NC + lax.axis_index("c")
    base = wid * B_PER_W

    pltpu.sync_copy(nid_hbm.at[pl.ds(base, B_PER_W)], idx_a)
    pltpu.sync_copy(nbr_hbm.at[pl.ds(base, B_PER_W)], idx_b)

    bias_copies = []
    for j in range(N_CHUNKS):
        sl = pl.ds(j * CHUNK, CHUNK)
        bias_copies.append(
            pltpu.async_copy(bias_hbm.at[idx_b.at[sl]], bias_v.at[sl], sem))

    lane = lax.iota(jnp.int32, L)

    def chunk_body(j, _):
        sl = pl.ds(j * CHUNK, CHUNK)
        ca = pltpu.async_copy(embp_hbm.at[idx_a.at[sl]], pair_a, sem)
        cb = pltpu.async_copy(embp_hbm.at[idx_b.at[sl]], pair_b, sem)
        ca.wait()
        cb.wait()

        def grp(g, _):
            i0 = j * CHUNK + g * L
            rows = g * L + lane

            def dim_body(d, acc):
                col = jnp.full((L,), 0, jnp.int32) + d
                va = plsc.load_gather(pair_a, [rows, col])
                vb = plsc.load_gather(pair_b, [rows, col])
                cols_a[d, pl.ds(i0, L)] = va
                cols_b[d, pl.ds(i0, L)] = vb
                return acc + va * vb
            acc = lax.fori_loop(0, EMB_DIM, dim_body,
                                jnp.zeros((L,), jnp.float32))
            acc_v[pl.ds(i0, L)] = acc
            return 0
        lax.fori_loop(0, GPC, grp, 0)
        return 0

    lax.fori_loop(0, N_CHUNKS, chunk_body, 0)

    for c in bias_copies:
        c.wait()

    def prob_grp(g, _):
        sl = pl.ds(g * L, L)
        score = acc_v[sl] + bias_v[sl]
        p = 1.0 / (1.0 + jnp.exp(-score))
        prob_v[sl] = jnp.minimum(jnp.maximum(p, 1e-5), 1.0)
        return 0
    lax.fori_loop(0, B_PER_W // L, prob_grp, 0)

    dst = pl.ds(base, B_PER_W)
    pltpu.sync_copy(cols_a, oa_hbm.at[:, dst])
    pltpu.sync_copy(cols_b, ob_hbm.at[:, dst])
    pltpu.sync_copy(prob_v, op_hbm.at[dst])


def _build():
    mesh = plsc.VectorSubcoreMesh(core_axis_name="c", subcore_axis_name="s")
    return pl.kernel(
        _sc_body,
        out_type=(
            jax.ShapeDtypeStruct((EMB_DIM, BATCH), jnp.float32),
            jax.ShapeDtypeStruct((EMB_DIM, BATCH), jnp.float32),
            jax.ShapeDtypeStruct((BATCH,), jnp.float32),
        ),
        mesh=mesh,
        scratch_types=[
            pltpu.VMEM((B_PER_W,), jnp.int32),
            pltpu.VMEM((B_PER_W,), jnp.int32),
            pltpu.VMEM((CHUNK, PADW), jnp.float32),
            pltpu.VMEM((CHUNK, PADW), jnp.float32),
            pltpu.VMEM((EMB_DIM, B_PER_W), jnp.float32),
            pltpu.VMEM((EMB_DIM, B_PER_W), jnp.float32),
            pltpu.VMEM((B_PER_W,), jnp.float32),
            pltpu.VMEM((B_PER_W,), jnp.float32),
            pltpu.VMEM((B_PER_W,), jnp.float32),
            pltpu.SemaphoreType.DMA,
        ],
        compiler_params=pltpu.CompilerParams(
            needs_layout_passes=False, use_tc_tiling_on_sc=True),
    )


def kernel(node_id, node_neighbor_id, embedding_matrix, bias_vector):
    k = _build()
    embp = jnp.pad(embedding_matrix, ((0, 0), (0, PADW - EMB_DIM)))
    oa_t, ob_t, prob = k(node_id, node_neighbor_id, embp, bias_vector)
    return oa_t.T, ob_t.T, prob
